# baseline (speedup 1.0000x reference)
import functools

import jax
import jax.numpy as jnp
from jax import lax
from jax.experimental import pallas as pl
from jax.experimental.pallas import tpu as pltpu
from jax.experimental.pallas import tpu_sc as plsc

N = 10000
E = 320000
D = 128
NC = 2
NS = 16
NW = NC * NS
EPT = E // NW
K = 80
NCH = EPT // K
NP = 10240
RPT = NP // NS
RCH = 128
L = 16


def _sc_mesh():
    return plsc.VectorSubcoreMesh(core_axis_name="c", subcore_axis_name="s")


def _scatter_stage(x, src, dst, zrows, zdeg1d):
    @functools.partial(
        pl.kernel,
        out_type=[
            jax.ShapeDtypeStruct((NC, NP, D), jnp.float32),
            jax.ShapeDtypeStruct((NW, NP), jnp.float32),
        ],
        mesh=_sc_mesh(),
        compiler_params=pltpu.CompilerParams(needs_layout_passes=False),
        scratch_types=[
            pltpu.VMEM((EPT,), jnp.int32),
            pltpu.VMEM((K,), jnp.int32),
            pltpu.VMEM((K,), jnp.int32),
            pltpu.VMEM((K, D), jnp.float32),
            pltpu.VMEM((K, D), jnp.float32),
            pltpu.VMEM((NP,), jnp.float32),
            pltpu.VMEM_SHARED((NP, D), jnp.float32),
            pltpu.SemaphoreType.DMA,
            pltpu.SemaphoreType.DMA,
            pltpu.SemaphoreType.DMA,
            pltpu.SemaphoreType.DMA,
        ],
    )
    def kern(x_hbm, src_hbm, dst_hbm, zrows_hbm, zdeg_hbm,
             aggp_hbm, degp_hbm,
             isrc_all, idst0, idst1, rows0, rows1, deg_v, agg_sh,
             gsem0, gsem1, ssem0, ssem1):
        c = lax.axis_index("c")
        s = lax.axis_index("s")
        wid = s * NC + c
        base = wid * EPT

        pltpu.sync_copy(zrows_hbm, agg_sh.at[pl.ds(s * RPT, RPT)])
        pltpu.sync_copy(zdeg_hbm, deg_v)
        plsc.subcore_barrier()

        pltpu.sync_copy(src_hbm.at[pl.ds(base, EPT)], isrc_all)

        ones_vec = jnp.ones((L,), jnp.float32)

        def fire_gather(n, rowsb, semb):
            idx = isrc_all.at[pl.ds(n * K, K)]
            pltpu.async_copy(x_hbm.at[idx], rowsb, semb)

        def wait_gather(rowsb, semb):
            idx = isrc_all.at[pl.ds(0, K)]
            pltpu.make_async_copy(x_hbm.at[idx], rowsb, semb).wait()

        def wait_scatter(rowsb, idstb, semb):
            pltpu.make_async_copy(rowsb, agg_sh.at[idstb], semb).wait()

        def deg_add(idstb):
            for v in range(K // L):
                idx = idstb[pl.ds(v * L, L)]
                plsc.addupdate_scatter(deg_v, [idx], ones_vec)

        fire_gather(0, rows0, gsem0)
        pltpu.sync_copy(dst_hbm.at[pl.ds(base, K)], idst0)
        pltpu.sync_copy(dst_hbm.at[pl.ds(base, K)], idst1)
        pltpu.sync_copy(zrows_hbm.at[pl.ds(0, K)], rows1)
        pltpu.async_copy(rows1, agg_sh.at[idst1], ssem1, add=True)

        def step(n, bufs):
            rowsA, idstA, gsemA, ssemA, rowsB, idstB, gsemB, ssemB = bufs
            wait_scatter(rowsB, idstB, ssemB)
            fire_gather(jnp.minimum(n + 1, NCH - 1), rowsB, gsemB)
            pltpu.sync_copy(dst_hbm.at[pl.ds(base + n * K, K)], idstA)
            wait_gather(rowsA, gsemA)
            pltpu.async_copy(rowsA, agg_sh.at[idstA], ssemA, add=True)
            deg_add(idstA)

        buf0 = (rows0, idst0, gsem0, ssem0)
        buf1 = (rows1, idst1, gsem1, ssem1)

        def pair(j, carry):
            step(2 * j, buf0 + buf1)
            step(2 * j + 1, buf1 + buf0)
            return carry

        lax.fori_loop(0, NCH // 2, pair, 0)
        wait_scatter(rows1, idst1, ssem1)
        wait_gather(rows0, gsem0)
        pltpu.sync_copy(dst_hbm.at[pl.ds(base + (NCH - 1) * K, K)], idst0)
        pltpu.async_copy(rows0, agg_sh.at[idst0], ssem0, add=True)
        deg_add(idst0)
        wait_scatter(rows0, idst0, ssem0)
        plsc.subcore_barrier()

        r0 = s * RPT
        pltpu.sync_copy(agg_sh.at[pl.ds(r0, RPT)],
                        aggp_hbm.at[c, pl.ds(r0, RPT)])
        pltpu.sync_copy(deg_v, degp_hbm.at[wid])

    return kern(x, src, dst, zrows, zdeg1d)


def _dense_kernel(aggp_ref, degp_ref, x_ref, wlt_ref, bl_ref, wrt_ref,
                  w1at_ref, w1bt_ref, b1_ref, a_ref, b_ref):
    agg = aggp_ref[0] + aggp_ref[1]
    deg = jnp.sum(degp_ref[...], axis=0)[:, None]
    mean = agg / jnp.maximum(deg, 1.0)
    h = mean @ wlt_ref[...] + bl_ref[...] + x_ref[...] @ wrt_ref[...]
    h = jnp.maximum(h, 0.0)
    a_ref[...] = (h @ w1at_ref[...]).astype(jnp.bfloat16)
    b_ref[...] = (h @ w1bt_ref[...] + b1_ref[...]).astype(jnp.bfloat16)


def _dense_stage(aggp, degp, x, wlt, bl2, wrt, w1at, w1bt, b12):
    nb = 8
    rb = NP // nb
    return pl.pallas_call(
        _dense_kernel,
        grid=(nb,),
        in_specs=[
            pl.BlockSpec((NC, rb, D), lambda i: (0, i, 0)),
            pl.BlockSpec((NW, rb), lambda i: (0, i)),
            pl.BlockSpec((rb, D), lambda i: (i, 0)),
            pl.BlockSpec((D, D), lambda i: (0, 0)),
            pl.BlockSpec((1, D), lambda i: (0, 0)),
            pl.BlockSpec((D, D), lambda i: (0, 0)),
            pl.BlockSpec((D, D), lambda i: (0, 0)),
            pl.BlockSpec((D, D), lambda i: (0, 0)),
            pl.BlockSpec((1, D), lambda i: (0, 0)),
        ],
        out_specs=[
            pl.BlockSpec((rb, D), lambda i: (i, 0)),
            pl.BlockSpec((rb, D), lambda i: (i, 0)),
        ],
        out_shape=[
            jax.ShapeDtypeStruct((NP, D), jnp.bfloat16),
            jax.ShapeDtypeStruct((NP, D), jnp.bfloat16),
        ],
    )(aggp, degp, x, wlt, bl2, wrt, w1at, w1bt, b12)


def _edge_stage(a, b, src, dst, w2v, b2v):
    @functools.partial(
        pl.kernel,
        out_type=jax.ShapeDtypeStruct((E,), jnp.float32),
        mesh=_sc_mesh(),
        compiler_params=pltpu.CompilerParams(
            needs_layout_passes=False, use_tc_tiling_on_sc=False),
        scratch_types=[
            pltpu.VMEM((EPT,), jnp.int32),
            pltpu.VMEM((EPT,), jnp.int32),
            pltpu.VMEM((K, D // 2), jnp.float32),
            pltpu.VMEM((K, D // 2), jnp.float32),
            pltpu.VMEM((K, D // 2), jnp.float32),
            pltpu.VMEM((K, D // 2), jnp.float32),
            pltpu.VMEM((D,), jnp.float32),
            pltpu.VMEM((L,), jnp.float32),
            pltpu.VMEM((EPT,), jnp.float32),
            pltpu.SemaphoreType.DMA,
            pltpu.SemaphoreType.DMA,
        ],
    )
    def kern(a_hbm, b_hbm, src_hbm, dst_hbm, w2_hbm, b2_hbm, out_hbm,
             isrc_all, idst_all, ar0, br0, ar1, br1,
             w2_v, b2_v, outb, sem0, sem1):
        c = lax.axis_index("c")
        s = lax.axis_index("s")
        wid = s * NC + c
        base = wid * EPT

        pltpu.sync_copy(w2_hbm, w2_v)
        pltpu.sync_copy(b2_hbm, b2_v)
        pltpu.sync_copy(src_hbm.at[pl.ds(base, EPT)], isrc_all)
        pltpu.sync_copy(dst_hbm.at[pl.ds(base, EPT)], idst_all)
        w2r = [w2_v[pl.ds(16 * j, 16)] for j in range(D // L)]
        b2vec = b2_v[...]
        lanes = lax.iota(jnp.int32, 16)

        def fire(n, arb, brb, semb):
            pltpu.async_copy(a_hbm.at[isrc_all.at[pl.ds(n * K, K)]], arb, semb)
            pltpu.async_copy(b_hbm.at[idst_all.at[pl.ds(n * K, K)]], brb, semb)

        def drain(arb, brb, semb):
            i0 = isrc_all.at[pl.ds(0, K)]
            pltpu.make_async_copy(a_hbm.at[i0], arb, semb).wait()
            pltpu.make_async_copy(b_hbm.at[i0], brb, semb).wait()

        def compute(arb, brb, n):
            @plsc.parallel_loop(0, K // L)
            def group(g):
                res = jnp.zeros((L,), jnp.float32)
                for l in range(L):
                    e = g * L + l
                    acc = None
                    for jj in range(D // (2 * L)):
                        aw = arb[e, pl.ds(16 * jj, 16)]
                        bw = brb[e, pl.ds(16 * jj, 16)]
                        ae, ao = plsc.unpack(
                            plsc.bitcast(aw, jnp.bfloat16),
                            format=plsc.PackFormat.INTERLEAVED,
                            preferred_element_type=jnp.float32)
                        be, bo = plsc.unpack(
                            plsc.bitcast(bw, jnp.bfloat16),
                            format=plsc.PackFormat.INTERLEAVED,
                            preferred_element_type=jnp.float32)
                        te = jnp.maximum(ae + be, 0.0) * w2r[2 * jj]
                        to = jnp.maximum(ao + bo, 0.0) * w2r[2 * jj + 1]
                        t = te + to
                        acc = t if acc is None else acc + t
                    total = plsc.cumsum(acc)[15]
                    res = jnp.where(lanes == l, total, res)
                outb[pl.ds(n * K + g * L, L)] = res + b2vec

        fire(0, ar0, br0, sem0)

        def pair(j, carry):
            n0 = 2 * j
            fire(n0 + 1, ar1, br1, sem1)
            drain(ar0, br0, sem0)
            compute(ar0, br0, n0)
            fire(jnp.minimum(n0 + 2, NCH - 1), ar0, br0, sem0)
            drain(ar1, br1, sem1)
            compute(ar1, br1, n0 + 1)
            return carry

        lax.fori_loop(0, NCH // 2, pair, 0)
        drain(ar0, br0, sem0)
        compute(ar0, br0, NCH - 1)
        pltpu.sync_copy(outb, out_hbm.at[pl.ds(base, EPT)])

    return kern(a, b, src, dst, w2v, b2v)


def kernel(x, edge_index, device, W_l, b_l, W_r, W1, b1, W2, b2):
    src = edge_index[0].astype(jnp.int32)
    dst = edge_index[1].astype(jnp.int32)

    zrows = jnp.zeros((RPT, D), jnp.float32)
    zdeg1d = jnp.zeros((NP,), jnp.float32)

    aggp, degp = _scatter_stage(x, src, dst, zrows, zdeg1d)

    xp = jnp.concatenate([x, jnp.zeros((NP - N, D), jnp.float32)], axis=0)
    a, b = _dense_stage(
        aggp, degp, xp,
        W_l.T, b_l.reshape(1, D), W_r.T,
        W1[:, :D].T, W1[:, D:].T, b1.reshape(1, D),
    )

    a2 = lax.bitcast_convert_type(a.reshape(NP, D // 2, 2), jnp.float32)
    b2_ = lax.bitcast_convert_type(b.reshape(NP, D // 2, 2), jnp.float32)
    w2blk = W2.reshape(D // 32, 16, 2)
    w2v = jnp.concatenate([w2blk[:, :, 0], w2blk[:, :, 1]], axis=1).reshape(D)
    b2v = jnp.full((L,), b2[0], jnp.float32)
    out = _edge_stage(a2, b2_, src, dst, w2v, b2v)
    return out.reshape(E, 1)


# speedup vs baseline: 2.2293x; 2.2293x over previous
import functools

import jax
import jax.numpy as jnp
from jax import lax
from jax.experimental import pallas as pl
from jax.experimental.pallas import tpu as pltpu
from jax.experimental.pallas import tpu_sc as plsc

N = 10000
E = 320000
D = 128
NC = 2
NS = 16
NW = NC * NS
EPT = E // NW
K = 80
NCH = EPT // K
NP = 10240
RPT = NP // NS
RCH = 128
L = 16


def _sc_mesh():
    return plsc.VectorSubcoreMesh(core_axis_name="c", subcore_axis_name="s")


def _scatter_stage(x, src, dst, zrows, zdeg1d):
    @functools.partial(
        pl.kernel,
        out_type=[
            jax.ShapeDtypeStruct((NC, NP, D), jnp.float32),
            jax.ShapeDtypeStruct((NW, NP), jnp.float32),
        ],
        mesh=_sc_mesh(),
        compiler_params=pltpu.CompilerParams(needs_layout_passes=False),
        scratch_types=[
            pltpu.VMEM((EPT,), jnp.int32),
            pltpu.VMEM((K,), jnp.int32),
            pltpu.VMEM((K,), jnp.int32),
            pltpu.VMEM((K, D), jnp.float32),
            pltpu.VMEM((K, D), jnp.float32),
            pltpu.VMEM((NP,), jnp.float32),
            pltpu.VMEM_SHARED((NP, D), jnp.float32),
            pltpu.SemaphoreType.DMA,
            pltpu.SemaphoreType.DMA,
            pltpu.SemaphoreType.DMA,
            pltpu.SemaphoreType.DMA,
        ],
    )
    def kern(x_hbm, src_hbm, dst_hbm, zrows_hbm, zdeg_hbm,
             aggp_hbm, degp_hbm,
             isrc_all, idst0, idst1, rows0, rows1, deg_v, agg_sh,
             gsem0, gsem1, ssem0, ssem1):
        c = lax.axis_index("c")
        s = lax.axis_index("s")
        wid = s * NC + c
        base = wid * EPT

        pltpu.sync_copy(zrows_hbm, agg_sh.at[pl.ds(s * RPT, RPT)])
        pltpu.sync_copy(zdeg_hbm, deg_v)
        plsc.subcore_barrier()

        pltpu.sync_copy(src_hbm.at[pl.ds(base, EPT)], isrc_all)

        ones_vec = jnp.ones((L,), jnp.float32)

        def fire_gather(n, rowsb, semb):
            idx = isrc_all.at[pl.ds(n * K, K)]
            pltpu.async_copy(x_hbm.at[idx], rowsb, semb)

        def wait_gather(rowsb, semb):
            idx = isrc_all.at[pl.ds(0, K)]
            pltpu.make_async_copy(x_hbm.at[idx], rowsb, semb).wait()

        def wait_scatter(rowsb, idstb, semb):
            pltpu.make_async_copy(rowsb, agg_sh.at[idstb], semb).wait()

        def deg_add(idstb):
            for v in range(K // L):
                idx = idstb[pl.ds(v * L, L)]
                plsc.addupdate_scatter(deg_v, [idx], ones_vec)

        fire_gather(0, rows0, gsem0)
        pltpu.sync_copy(dst_hbm.at[pl.ds(base, K)], idst0)
        pltpu.sync_copy(dst_hbm.at[pl.ds(base, K)], idst1)
        pltpu.sync_copy(zrows_hbm.at[pl.ds(0, K)], rows1)
        pltpu.async_copy(rows1, agg_sh.at[idst1], ssem1, add=True)

        def step(n, bufs):
            rowsA, idstA, gsemA, ssemA, rowsB, idstB, gsemB, ssemB = bufs
            wait_scatter(rowsB, idstB, ssemB)
            fire_gather(jnp.minimum(n + 1, NCH - 1), rowsB, gsemB)
            pltpu.sync_copy(dst_hbm.at[pl.ds(base + n * K, K)], idstA)
            wait_gather(rowsA, gsemA)
            pltpu.async_copy(rowsA, agg_sh.at[idstA], ssemA, add=True)
            deg_add(idstA)

        buf0 = (rows0, idst0, gsem0, ssem0)
        buf1 = (rows1, idst1, gsem1, ssem1)

        def pair(j, carry):
            step(2 * j, buf0 + buf1)
            step(2 * j + 1, buf1 + buf0)
            return carry

        lax.fori_loop(0, NCH // 2, pair, 0)
        wait_scatter(rows1, idst1, ssem1)
        wait_gather(rows0, gsem0)
        pltpu.sync_copy(dst_hbm.at[pl.ds(base + (NCH - 1) * K, K)], idst0)
        pltpu.async_copy(rows0, agg_sh.at[idst0], ssem0, add=True)
        deg_add(idst0)
        wait_scatter(rows0, idst0, ssem0)
        plsc.subcore_barrier()

        r0 = s * RPT
        pltpu.sync_copy(agg_sh.at[pl.ds(r0, RPT)],
                        aggp_hbm.at[c, pl.ds(r0, RPT)])
        pltpu.sync_copy(deg_v, degp_hbm.at[wid])

    return kern(x, src, dst, zrows, zdeg1d)


def _dense_kernel(aggp_ref, degp_ref, x_ref, wlt_ref, bl_ref, wrt_ref,
                  w1at_ref, w1bt_ref, b1_ref, a_ref, b_ref):
    agg = aggp_ref[0] + aggp_ref[1]
    deg = jnp.sum(degp_ref[...], axis=0)[:, None]
    mean = agg / jnp.maximum(deg, 1.0)
    h = mean @ wlt_ref[...] + bl_ref[...] + x_ref[...] @ wrt_ref[...]
    h = jnp.maximum(h, 0.0)
    a_ref[...] = h @ w1at_ref[...]
    b_ref[...] = h @ w1bt_ref[...] + b1_ref[...]


def _dense_stage(aggp, degp, x, wlt, bl2, wrt, w1at, w1bt, b12):
    nb = 8
    rb = NP // nb
    return pl.pallas_call(
        _dense_kernel,
        grid=(nb,),
        in_specs=[
            pl.BlockSpec((NC, rb, D), lambda i: (0, i, 0)),
            pl.BlockSpec((NW, rb), lambda i: (0, i)),
            pl.BlockSpec((rb, D), lambda i: (i, 0)),
            pl.BlockSpec((D, D), lambda i: (0, 0)),
            pl.BlockSpec((1, D), lambda i: (0, 0)),
            pl.BlockSpec((D, D), lambda i: (0, 0)),
            pl.BlockSpec((D, D), lambda i: (0, 0)),
            pl.BlockSpec((D, D), lambda i: (0, 0)),
            pl.BlockSpec((1, D), lambda i: (0, 0)),
        ],
        out_specs=[
            pl.BlockSpec((rb, D), lambda i: (i, 0)),
            pl.BlockSpec((rb, D), lambda i: (i, 0)),
        ],
        out_shape=[
            jax.ShapeDtypeStruct((NP, D), jnp.float32),
            jax.ShapeDtypeStruct((NP, D), jnp.float32),
        ],
    )(aggp, degp, x, wlt, bl2, wrt, w1at, w1bt, b12)


def _edge_stage(a, b, src, dst, w2v, b2v):
    @functools.partial(
        pl.kernel,
        out_type=jax.ShapeDtypeStruct((E,), jnp.float32),
        mesh=_sc_mesh(),
        compiler_params=pltpu.CompilerParams(needs_layout_passes=False),
        scratch_types=[
            pltpu.VMEM((EPT,), jnp.int32),
            pltpu.VMEM((EPT,), jnp.int32),
            pltpu.VMEM((K, D), jnp.float32),
            pltpu.VMEM((K, D), jnp.float32),
            pltpu.VMEM((K, D), jnp.float32),
            pltpu.VMEM((K, D), jnp.float32),
            pltpu.VMEM((D,), jnp.float32),
            pltpu.VMEM((L,), jnp.float32),
            pltpu.VMEM((EPT,), jnp.float32),
            pltpu.SemaphoreType.DMA,
            pltpu.SemaphoreType.DMA,
        ],
    )
    def kern(a_hbm, b_hbm, src_hbm, dst_hbm, w2_hbm, b2_hbm, out_hbm,
             isrc_all, idst_all, ar0, br0, ar1, br1,
             w2_v, b2_v, outb, sem0, sem1):
        c = lax.axis_index("c")
        s = lax.axis_index("s")
        wid = s * NC + c
        base = wid * EPT

        pltpu.sync_copy(w2_hbm, w2_v)
        pltpu.sync_copy(b2_hbm, b2_v)
        pltpu.sync_copy(src_hbm.at[pl.ds(base, EPT)], isrc_all)
        pltpu.sync_copy(dst_hbm.at[pl.ds(base, EPT)], idst_all)
        w2r = [w2_v[pl.ds(16 * j, 16)] for j in range(D // L)]
        b2vec = b2_v[...]
        lanes = lax.iota(jnp.int32, 16)

        def fire(n, arb, brb, semb):
            pltpu.async_copy(a_hbm.at[isrc_all.at[pl.ds(n * K, K)]], arb, semb)
            pltpu.async_copy(b_hbm.at[idst_all.at[pl.ds(n * K, K)]], brb, semb)

        def drain(arb, brb, semb):
            i0 = isrc_all.at[pl.ds(0, K)]
            pltpu.make_async_copy(a_hbm.at[i0], arb, semb).wait()
            pltpu.make_async_copy(b_hbm.at[i0], brb, semb).wait()

        def compute(arb, brb, n):
            @plsc.parallel_loop(0, K // L)
            def group(g):
                res = jnp.zeros((L,), jnp.float32)
                for l in range(L):
                    e = g * L + l
                    acc = None
                    for j in range(D // L):
                        av = arb[e, pl.ds(16 * j, 16)]
                        bv = brb[e, pl.ds(16 * j, 16)]
                        t = jnp.maximum(av + bv, 0.0) * w2r[j]
                        acc = t if acc is None else acc + t
                    total = plsc.cumsum(acc)[15]
                    res = jnp.where(lanes == l, total, res)
                outb[pl.ds(n * K + g * L, L)] = res + b2vec

        fire(0, ar0, br0, sem0)

        def pair(j, carry):
            n0 = 2 * j
            fire(n0 + 1, ar1, br1, sem1)
            drain(ar0, br0, sem0)
            compute(ar0, br0, n0)
            fire(jnp.minimum(n0 + 2, NCH - 1), ar0, br0, sem0)
            drain(ar1, br1, sem1)
            compute(ar1, br1, n0 + 1)
            return carry

        lax.fori_loop(0, NCH // 2, pair, 0)
        drain(ar0, br0, sem0)
        compute(ar0, br0, NCH - 1)
        pltpu.sync_copy(outb, out_hbm.at[pl.ds(base, EPT)])

    return kern(a, b, src, dst, w2v, b2v)


def kernel(x, edge_index, device, W_l, b_l, W_r, W1, b1, W2, b2):
    src = edge_index[0].astype(jnp.int32)
    dst = edge_index[1].astype(jnp.int32)

    zrows = jnp.zeros((RPT, D), jnp.float32)
    zdeg1d = jnp.zeros((NP,), jnp.float32)

    aggp, degp = _scatter_stage(x, src, dst, zrows, zdeg1d)

    xp = jnp.concatenate([x, jnp.zeros((NP - N, D), jnp.float32)], axis=0)
    a, b = _dense_stage(
        aggp, degp, xp,
        W_l.T, b_l.reshape(1, D), W_r.T,
        W1[:, :D].T, W1[:, D:].T, b1.reshape(1, D),
    )

    w2v = W2.reshape(D)
    b2v = jnp.full((L,), b2[0], jnp.float32)
    out = _edge_stage(a, b, src, dst, w2v, b2v)
    return out.reshape(E, 1)
